# baseline (speedup 1.0000x reference)
import functools

import jax
import jax.numpy as jnp
from jax import lax
from jax.experimental import pallas as pl
from jax.experimental.pallas import tpu as pltpu
from jax.experimental.pallas import tpu_sc as plsc


def _leaky(t):
    return jnp.where(t >= 0, t, 0.2 * t)


def _node_tables_body(x_ref, w_ref, b_ref, xu_ref, xv_ref, xa_ref, xb_ref):
    p = (jnp.dot(x_ref[...], w_ref[...], preferred_element_type=jnp.float32)
         + b_ref[...])
    d = x_ref.shape[1]
    xu_ref[...] = p[:, :d]
    xv_ref[...] = p[:, d:2 * d]
    xa_ref[...] = p[:, 2 * d:3 * d]
    xb_ref[...] = p[:, 3 * d:]


def _node_tables(x, wcat, bcat, blk):
    n, d = x.shape
    dout = wcat.shape[1]
    out = jax.ShapeDtypeStruct((n, d), jnp.float32)
    return pl.pallas_call(
        _node_tables_body,
        grid=(n // blk,),
        in_specs=[
            pl.BlockSpec((blk, d), lambda i: (i, 0)),
            pl.BlockSpec((d, dout), lambda i: (0, 0)),
            pl.BlockSpec((1, dout), lambda i: (0, 0)),
        ],
        out_specs=[pl.BlockSpec((blk, d), lambda i: (i, 0))] * 4,
        out_shape=[out, out, out, out],
    )(x, wcat, bcat.reshape(1, -1))


def _sc_edge_pass(e_hat, edge_index, xv, xa, xb, n, e, d, w):
    per_tile = e // 16
    chunks = per_tile // w
    nblk = n // w
    blk_iters = -(-nblk // 16)

    mesh = plsc.VectorSubcoreMesh(core_axis_name="c", subcore_axis_name="s")

    @functools.partial(
        pl.kernel,
        out_type=(
            jax.ShapeDtypeStruct((e, d), jnp.float32),
            jax.ShapeDtypeStruct((2 * n, d), jnp.float32),
        ),
        mesh=mesh,
        scratch_types=[
            pltpu.VMEM((w,), jnp.int32),
            pltpu.VMEM((w,), jnp.int32),
            pltpu.VMEM((w, d), jnp.float32),
            pltpu.VMEM((w, d), jnp.float32),
            pltpu.VMEM((w, d), jnp.float32),
            pltpu.VMEM_SHARED((n, d), jnp.float32),
            pltpu.SemaphoreType.DMA,
            pltpu.SemaphoreType.DMA,
            pltpu.SemaphoreType.DMA,
            pltpu.SemaphoreType.DMA,
        ],
    )
    def sc_edge(
        ehat_hbm, src_hbm, dst_hbm, xv_hbm, xa_hbm, xb_hbm, zeros_hbm,
        xab_out, acc_out,
        src_v, dst_v, ehat_v, t1_v, t2_v, sacc, sem1, sem2, sem3, sem4,
    ):
        c = lax.axis_index("c")
        s = lax.axis_index("s")
        coff = c * n

        for b in range(blk_iters):
            bid = 16 * b + s

            @pl.when(bid < nblk)
            def _zero_blk():
                pltpu.sync_copy(zeros_hbm.at[pl.ds(bid * w, w)],
                                sacc.at[pl.ds(bid * w, w)])

        plsc.subcore_barrier()

        nj = d // 16
        ur = 8

        def drain_scatter():
            @pl.when(c == 0)
            def _d0():
                pltpu.make_async_copy(ehat_v, sacc.at[src_v], sem3).wait()
                pltpu.make_async_copy(t1_v, xab_out.at[pl.ds(0, w)], sem4).wait()

            @pl.when(c == 1)
            def _d1():
                pltpu.make_async_copy(t1_v, sacc.at[src_v], sem3).wait()

        def body(k, carry):
            @pl.when(k > 0)
            def _drain_prev():
                drain_scatter()

            base = s * per_tile + k * w
            pltpu.sync_copy(src_hbm.at[pl.ds(base, w)], src_v)
            pltpu.sync_copy(dst_hbm.at[pl.ds(base, w)], dst_v)

            @pl.when(c == 0)
            def _fire0():
                pltpu.async_copy(xa_hbm.at[src_v], t1_v, sem1)
                pltpu.async_copy(xb_hbm.at[dst_v], t2_v, sem2)

            @pl.when(c == 1)
            def _fire1():
                pltpu.async_copy(xv_hbm.at[dst_v], t1_v, sem1)

            pltpu.sync_copy(ehat_hbm.at[pl.ds(base, w)], ehat_v)

            def exp_row(r, carry2):
                for u in range(ur):
                    for j in range(nj):
                        sl = pl.ds(16 * j, 16)
                        ehat_v[r * ur + u, sl] = jnp.exp(ehat_v[r * ur + u, sl])
                return carry2

            lax.fori_loop(0, w // ur, exp_row, 0)

            @pl.when(c == 0)
            def _den_xab():
                pltpu.async_copy(ehat_v, sacc.at[src_v], sem3, add=True)
                pltpu.make_async_copy(xa_hbm.at[src_v], t1_v, sem1).wait()
                pltpu.make_async_copy(xb_hbm.at[dst_v], t2_v, sem2).wait()

                def add_row(r, carry2):
                    for u in range(ur):
                        for j in range(nj):
                            sl = pl.ds(16 * j, 16)
                            t1_v[r * ur + u, sl] = (t1_v[r * ur + u, sl]
                                                    + t2_v[r * ur + u, sl])
                    return carry2

                lax.fori_loop(0, w // ur, add_row, 0)
                pltpu.async_copy(t1_v, xab_out.at[pl.ds(base, w)], sem4)

            @pl.when(c == 1)
            def _num():
                pltpu.make_async_copy(xv_hbm.at[dst_v], t1_v, sem1).wait()

                def mul_row(r, carry2):
                    for u in range(ur):
                        for j in range(nj):
                            sl = pl.ds(16 * j, 16)
                            t1_v[r * ur + u, sl] = (t1_v[r * ur + u, sl]
                                                    * ehat_v[r * ur + u, sl])
                    return carry2

                lax.fori_loop(0, w // ur, mul_row, 0)
                pltpu.async_copy(t1_v, sacc.at[src_v], sem3, add=True)

            return carry

        lax.fori_loop(0, chunks, body, 0)
        drain_scatter()
        plsc.subcore_barrier()

        for b in range(blk_iters):
            bid = 16 * b + s

            @pl.when(bid < nblk)
            def _read_blk():
                pltpu.sync_copy(sacc.at[pl.ds(bid * w, w)],
                                acc_out.at[pl.ds(coff + bid * w, w)])

    src = edge_index[0]
    dst = edge_index[1]
    zeros_nd = jnp.zeros((n, d), jnp.float32)
    return sc_edge(e_hat, src, dst, xv, xa, xb, zeros_nd)


def _edge_update_body(eh_ref, xab_ref, ct_ref, o_ref):
    eh = eh_ref[...]
    t = jnp.dot(eh, ct_ref[...], preferred_element_type=jnp.float32)
    t = t + xab_ref[...]
    o_ref[...] = eh + _leaky(t)


def _edge_update(e_hat, xab, ct, e, d, blk):
    return pl.pallas_call(
        _edge_update_body,
        grid=(e // blk,),
        in_specs=[
            pl.BlockSpec((blk, d), lambda i: (i, 0)),
            pl.BlockSpec((blk, d), lambda i: (i, 0)),
            pl.BlockSpec((d, d), lambda i: (0, 0)),
        ],
        out_specs=pl.BlockSpec((blk, d), lambda i: (i, 0)),
        out_shape=jax.ShapeDtypeStruct((e, d), jnp.float32),
    )(e_hat, xab, ct)


def _node_update_body(x_ref, xu_ref, den_ref, num_ref, o_ref):
    den = den_ref[...]
    y = jnp.where(den > 0, num_ref[...] / den, 0.0)
    o_ref[...] = x_ref[...] + _leaky(xu_ref[...] + y)


def _node_update(x, xu, acc, n, d, blk):
    steps = n // blk
    return pl.pallas_call(
        _node_update_body,
        grid=(steps,),
        in_specs=[
            pl.BlockSpec((blk, d), lambda i: (i, 0)),
            pl.BlockSpec((blk, d), lambda i: (i, 0)),
            pl.BlockSpec((blk, d), lambda i: (i, 0)),
            pl.BlockSpec((blk, d), lambda i: (steps + i, 0)),
        ],
        out_specs=pl.BlockSpec((blk, d), lambda i: (i, 0)),
        out_shape=jax.ShapeDtypeStruct((n, d), jnp.float32),
    )(x, xu, acc, acc)


def kernel(x, edge_index, e_hat, U_W, U_b, V_W, V_b, A_W, A_b, B_W, B_b, C_W, C_b):
    n, d = x.shape
    e = e_hat.shape[0]

    wcat = jnp.concatenate([U_W.T, V_W.T, A_W.T, B_W.T], axis=1)
    bcat = jnp.concatenate([U_b, V_b, A_b + B_b + C_b, jnp.zeros_like(B_b)], axis=0)
    xu, xv, xa, xb = _node_tables(x, wcat, bcat, blk=2000)

    xab, acc = _sc_edge_pass(e_hat, edge_index, xv, xa, xb, n, e, d, w=80)

    e_new = _edge_update(e_hat, xab, C_W.T, e, d, blk=2000)
    x_new = _node_update(x, xu, acc, n, d, blk=2000)
    return (x_new, e_new)


# speedup vs baseline: 1.1040x; 1.1040x over previous
import functools

import jax
import jax.numpy as jnp
from jax import lax
from jax.experimental import pallas as pl
from jax.experimental.pallas import tpu as pltpu
from jax.experimental.pallas import tpu_sc as plsc


def _leaky(t):
    return jnp.where(t >= 0, t, 0.2 * t)


def _node_tables_body(x_ref, w_ref, b_ref, xu_ref, xv_ref, xa_ref, xb_ref):
    p = (jnp.dot(x_ref[...], w_ref[...], preferred_element_type=jnp.float32)
         + b_ref[...])
    d = x_ref.shape[1]
    xu_ref[...] = p[:, :d]
    xv_ref[...] = p[:, d:2 * d]
    xa_ref[...] = p[:, 2 * d:3 * d]
    xb_ref[...] = p[:, 3 * d:]


def _node_tables(x, wcat, bcat, blk):
    n, d = x.shape
    dout = wcat.shape[1]
    out = jax.ShapeDtypeStruct((n, d), jnp.float32)
    return pl.pallas_call(
        _node_tables_body,
        grid=(n // blk,),
        in_specs=[
            pl.BlockSpec((blk, d), lambda i: (i, 0)),
            pl.BlockSpec((d, dout), lambda i: (0, 0)),
            pl.BlockSpec((1, dout), lambda i: (0, 0)),
        ],
        out_specs=[pl.BlockSpec((blk, d), lambda i: (i, 0))] * 4,
        out_shape=[out, out, out, out],
    )(x, wcat, bcat.reshape(1, -1))


def _sc_edge_pass(e_hat, edge_index, xv, xa, xb, n, e, d, w):
    per_tile = e // 16
    chunks = per_tile // w
    nblk = n // w
    blk_iters = -(-nblk // 16)

    mesh = plsc.VectorSubcoreMesh(core_axis_name="c", subcore_axis_name="s")

    @functools.partial(
        pl.kernel,
        out_type=(
            jax.ShapeDtypeStruct((e, d), jnp.float32),
            jax.ShapeDtypeStruct((2 * n, d), jnp.float32),
        ),
        mesh=mesh,
        scratch_types=[
            pltpu.VMEM((8, w), jnp.int32),
            pltpu.VMEM((8, w), jnp.int32),
            pltpu.VMEM((w, d), jnp.float32),
            pltpu.VMEM((w, d), jnp.float32),
            pltpu.VMEM((w, d), jnp.float32),
            pltpu.VMEM_SHARED((n, d), jnp.float32),
            pltpu.SemaphoreType.DMA,
            pltpu.SemaphoreType.DMA,
            pltpu.SemaphoreType.DMA,
            pltpu.SemaphoreType.DMA,
        ],
    )
    def sc_edge(
        ehat_hbm, src_hbm, dst_hbm, xv_hbm, xa_hbm, xb_hbm, zeros_hbm,
        xab_out, acc_out,
        src_v, dst_v, ehat_v, t1_v, t2_v, sacc, sem1, sem2, sem3, sem4,
    ):
        c = lax.axis_index("c")
        s = lax.axis_index("s")
        coff = c * n

        for b in range(blk_iters):
            bid = 16 * b + s

            @pl.when(bid < nblk)
            def _zero_blk():
                pltpu.sync_copy(zeros_hbm.at[pl.ds(bid * w, w)],
                                sacc.at[pl.ds(bid * w, w)])

        plsc.subcore_barrier()

        nj = d // 16
        ur = 8
        gsz = 8
        ngroups = e // (w * gsz)
        giters = -(-ngroups // 16)

        def drain_scatter(j):
            sj = src_v.at[j]

            @pl.when(c == 0)
            def _d0():
                pltpu.make_async_copy(ehat_v, sacc.at[sj], sem3).wait()
                pltpu.make_async_copy(t1_v, xab_out.at[pl.ds(0, w)], sem4).wait()

            @pl.when(c == 1)
            def _d1():
                pltpu.make_async_copy(t1_v, sacc.at[sj], sem3).wait()

        def chunk_step(base, sj, dj):
            @pl.when(c == 0)
            def _fire0():
                pltpu.async_copy(xa_hbm.at[sj], t1_v, sem1)
                pltpu.async_copy(xb_hbm.at[dj], t2_v, sem2)

            @pl.when(c == 1)
            def _fire1():
                pltpu.async_copy(xv_hbm.at[dj], t1_v, sem1)

            pltpu.sync_copy(ehat_hbm.at[pl.ds(base, w)], ehat_v)

            def exp_row(r, carry2):
                for u in range(ur):
                    for j in range(nj):
                        sl = pl.ds(16 * j, 16)
                        ehat_v[r * ur + u, sl] = jnp.exp(ehat_v[r * ur + u, sl])
                return carry2

            lax.fori_loop(0, w // ur, exp_row, 0)

            @pl.when(c == 0)
            def _den_xab():
                pltpu.async_copy(ehat_v, sacc.at[sj], sem3, add=True)
                pltpu.make_async_copy(xa_hbm.at[sj], t1_v, sem1).wait()
                pltpu.make_async_copy(xb_hbm.at[dj], t2_v, sem2).wait()

                def add_row(r, carry2):
                    for u in range(ur):
                        for j in range(nj):
                            sl = pl.ds(16 * j, 16)
                            t1_v[r * ur + u, sl] = (t1_v[r * ur + u, sl]
                                                    + t2_v[r * ur + u, sl])
                    return carry2

                lax.fori_loop(0, w // ur, add_row, 0)
                pltpu.async_copy(t1_v, xab_out.at[pl.ds(base, w)], sem4)

            @pl.when(c == 1)
            def _num():
                pltpu.make_async_copy(xv_hbm.at[dj], t1_v, sem1).wait()

                def mul_row(r, carry2):
                    for u in range(ur):
                        for j in range(nj):
                            sl = pl.ds(16 * j, 16)
                            t1_v[r * ur + u, sl] = (t1_v[r * ur + u, sl]
                                                    * ehat_v[r * ur + u, sl])
                    return carry2

                lax.fori_loop(0, w // ur, mul_row, 0)
                pltpu.async_copy(t1_v, sacc.at[sj], sem3, add=True)

        def body(gi, carry):
            g = gi * 16 + s

            @pl.when(g < ngroups)
            def _grp():
                @pl.when(gi > 0)
                def _drain_prev_group():
                    drain_scatter(gsz - 1)

                pltpu.sync_copy(src_hbm.at[g], src_v)
                pltpu.sync_copy(dst_hbm.at[g], dst_v)

                for j in range(gsz):
                    if j > 0:
                        drain_scatter(j - 1)
                    chunk_step((g * gsz + j) * w, src_v.at[j], dst_v.at[j])

            return carry

        lax.fori_loop(0, giters, body, 0)
        drain_scatter(gsz - 1)
        plsc.subcore_barrier()

        for b in range(blk_iters):
            bid = 16 * b + s

            @pl.when(bid < nblk)
            def _read_blk():
                pltpu.sync_copy(sacc.at[pl.ds(bid * w, w)],
                                acc_out.at[pl.ds(coff + bid * w, w)])

    src = edge_index[0].reshape(e // (w * 8), 8, w)
    dst = edge_index[1].reshape(e // (w * 8), 8, w)
    zeros_nd = jnp.zeros((n, d), jnp.float32)
    return sc_edge(e_hat, src, dst, xv, xa, xb, zeros_nd)


def _edge_update_body(eh_ref, xab_ref, ct_ref, o_ref):
    eh = eh_ref[...]
    t = jnp.dot(eh, ct_ref[...], preferred_element_type=jnp.float32)
    t = t + xab_ref[...]
    o_ref[...] = eh + _leaky(t)


def _edge_update(e_hat, xab, ct, e, d, blk):
    return pl.pallas_call(
        _edge_update_body,
        grid=(e // blk,),
        in_specs=[
            pl.BlockSpec((blk, d), lambda i: (i, 0)),
            pl.BlockSpec((blk, d), lambda i: (i, 0)),
            pl.BlockSpec((d, d), lambda i: (0, 0)),
        ],
        out_specs=pl.BlockSpec((blk, d), lambda i: (i, 0)),
        out_shape=jax.ShapeDtypeStruct((e, d), jnp.float32),
    )(e_hat, xab, ct)


def _node_update_body(x_ref, xu_ref, den_ref, num_ref, o_ref):
    den = den_ref[...]
    y = jnp.where(den > 0, num_ref[...] / den, 0.0)
    o_ref[...] = x_ref[...] + _leaky(xu_ref[...] + y)


def _node_update(x, xu, acc, n, d, blk):
    steps = n // blk
    return pl.pallas_call(
        _node_update_body,
        grid=(steps,),
        in_specs=[
            pl.BlockSpec((blk, d), lambda i: (i, 0)),
            pl.BlockSpec((blk, d), lambda i: (i, 0)),
            pl.BlockSpec((blk, d), lambda i: (i, 0)),
            pl.BlockSpec((blk, d), lambda i: (steps + i, 0)),
        ],
        out_specs=pl.BlockSpec((blk, d), lambda i: (i, 0)),
        out_shape=jax.ShapeDtypeStruct((n, d), jnp.float32),
    )(x, xu, acc, acc)


def kernel(x, edge_index, e_hat, U_W, U_b, V_W, V_b, A_W, A_b, B_W, B_b, C_W, C_b):
    n, d = x.shape
    e = e_hat.shape[0]

    wcat = jnp.concatenate([U_W.T, V_W.T, A_W.T, B_W.T], axis=1)
    bcat = jnp.concatenate([U_b, V_b, A_b + B_b + C_b, jnp.zeros_like(B_b)], axis=0)
    xu, xv, xa, xb = _node_tables(x, wcat, bcat, blk=2000)

    xab, acc = _sc_edge_pass(e_hat, edge_index, xv, xa, xb, n, e, d, w=80)

    e_new = _edge_update(e_hat, xab, C_W.T, e, d, blk=2000)
    x_new = _node_update(x, xu, acc, n, d, blk=2000)
    return (x_new, e_new)
